# Initial kernel scaffold; baseline (speedup 1.0000x reference)
import jax
import jax.numpy as jnp
from jax.experimental import pallas as pl


def kernel(in_tensor, conditioning_code, tables):
    raise NotImplementedError("write your pallas kernel here")


# speedup vs baseline: 1.5796x; 1.5796x over previous
import functools

import jax
import jax.numpy as jnp
import numpy as np
from jax import lax
from jax.experimental import pallas as pl
from jax.experimental.pallas import tpu as pltpu
from jax.experimental.pallas import tpu_sc as plsc

N_POINTS = 65536
N_ENC = 4
N_LEVELS = 16
F_PER_LEVEL = 2
LOG2_T = 19
T = 1 << LOG2_T
BASE_RES = 16
PER_LEVEL_SCALE = 1.4472692012786865
P1_I32 = int(np.uint32(2654435761).view(np.int32))
P2_I32 = int(np.uint32(805459861).view(np.int32))

OUT_F = N_LEVELS * F_PER_LEVEL
ROW_F = N_ENC * F_PER_LEVEL

NW = 32
PT = N_POINTS // NW
P = 512
G = P // 16
CH = PT // P

_LEVELS = []
for _l in range(N_LEVELS):
    _scale = BASE_RES * (PER_LEVEL_SCALE ** _l) - 1.0
    _res = int(np.ceil(_scale)) + 1
    _LEVELS.append((np.float32(_scale), _res, (_res ** 3) <= T))


def _full(v, dtype=jnp.int32):
    return jnp.full((16,), v, dtype)


def _lane():
    return lax.iota(jnp.int32, 16)


def _load_pos(coords_v, p, scale):
    pos_i, frac = [], []
    for d in range(3):
        x = plsc.load_gather(coords_v, [p, _full(d)])
        px = x * scale + np.float32(0.5)
        pi = px.astype(jnp.int32)
        frac.append(px - pi.astype(jnp.float32))
        pos_i.append(pi)
    return pos_i, frac


def _idx_pass(l, coords_v, idx_ref):
    scale, res, dense = _LEVELS[l]
    lbase = l * T

    def body(g, carry):
        p = g * 16 + _lane()
        pos_i, _ = _load_pos(coords_v, p, scale)
        if dense:
            xs = (pos_i[0], pos_i[0] + 1)
            ys = (pos_i[1] * res, (pos_i[1] + 1) * res)
            zs = (pos_i[2] * (res * res), (pos_i[2] + 1) * (res * res))
        else:
            xs = (pos_i[0], pos_i[0] + 1)
            ys = (pos_i[1] * P1_I32, pos_i[1] * P1_I32 + P1_I32)
            zs = (pos_i[2] * P2_I32, pos_i[2] * P2_I32 + P2_I32)
        base = g * 128
        for c in range(8):
            ox, oy, oz = c & 1, (c >> 1) & 1, (c >> 2) & 1
            if dense:
                h = xs[ox] + ys[oy] + zs[oz]
            else:
                h = (xs[ox] ^ ys[oy]) ^ zs[oz]
            gidx = (h & (T - 1)) + lbase
            idx_ref[pl.ds(base + c * 16, 16)] = gidx
        return carry

    lax.fori_loop(0, G, body, 0)


def _blend_pass(l, coords_v, code_v, dst_ref, out_v):
    scale, _, _ = _LEVELS[l]

    def body(g, carry):
        p = g * 16 + _lane()
        _, frac = _load_pos(coords_v, p, scale)
        one = np.float32(1.0)
        wx = (one - frac[0], frac[0])
        wy = (one - frac[1], frac[1])
        wz = (one - frac[2], frac[2])
        wxy = tuple(wx[ox] * wy[oy] for oy in range(2) for ox in range(2))
        code = [plsc.load_gather(code_v, [p, _full(e)]) for e in range(N_ENC)]
        acc0 = jnp.zeros((16,), jnp.float32)
        acc1 = jnp.zeros((16,), jnp.float32)
        base = g * 128
        for c in range(8):
            ox, oy, oz = c & 1, (c >> 1) & 1, (c >> 2) & 1
            rb = base + c * 16 + _lane()
            r = [plsc.load_gather(dst_ref, [rb, _full(col)])
                 for col in range(ROW_F)]
            b0 = (code[0] * r[0] + code[1] * r[2]) + (code[2] * r[4] + code[3] * r[6])
            b1 = (code[0] * r[1] + code[1] * r[3]) + (code[2] * r[5] + code[3] * r[7])
            w = wxy[oy * 2 + ox] * wz[oz]
            acc0 = acc0 + w * b0
            acc1 = acc1 + w * b1
        plsc.store_scatter(out_v, [p, _full(2 * l)], acc0)
        plsc.store_scatter(out_v, [p, _full(2 * l + 1)], acc1)
        return carry

    lax.fori_loop(0, G, body, 0)


def _hash_ensemble_sc(xyz, code, table, out,
                      coords_v, code_v, idx_a, idx_b, dst_a, dst_b, out_v,
                      sem_a, sem_b):
    wid = lax.axis_index("s") * 2 + lax.axis_index("c")
    idx_bufs = (idx_a, idx_b)
    dst_bufs = (dst_a, dst_b)
    sems = (sem_a, sem_b)

    def chunk_body(ch, carry):
        base = wid * PT + ch * P
        pltpu.sync_copy(xyz.at[pl.ds(base, P), :], coords_v)
        pltpu.sync_copy(code.at[pl.ds(base, P), :], code_v)
        _idx_pass(0, coords_v, idx_bufs[0])
        pltpu.make_async_copy(table.at[idx_bufs[0]], dst_bufs[0], sems[0]).start()
        for l in range(N_LEVELS):
            cur = l & 1
            nxt = 1 - cur
            if l + 1 < N_LEVELS:
                _idx_pass(l + 1, coords_v, idx_bufs[nxt])
                pltpu.make_async_copy(
                    table.at[idx_bufs[nxt]], dst_bufs[nxt], sems[nxt]).start()
            pltpu.make_async_copy(
                table.at[idx_bufs[cur]], dst_bufs[cur], sems[cur]).wait()
            _blend_pass(l, coords_v, code_v, dst_bufs[cur], out_v)
        pltpu.sync_copy(out_v, out.at[pl.ds(base, P), :])
        return carry

    lax.fori_loop(0, CH, chunk_body, 0)


@functools.cache
def _build_sc_kernel():
    return pl.kernel(
        _hash_ensemble_sc,
        out_type=jax.ShapeDtypeStruct((N_POINTS, OUT_F), jnp.float32),
        mesh=plsc.VectorSubcoreMesh(core_axis_name="c", subcore_axis_name="s"),
        compiler_params=pltpu.CompilerParams(
            needs_layout_passes=False, use_tc_tiling_on_sc=False),
        scratch_types=[
            pltpu.VMEM((P, 3), jnp.float32),
            pltpu.VMEM((P, N_ENC), jnp.float32),
            pltpu.VMEM((P * 8,), jnp.int32),
            pltpu.VMEM((P * 8,), jnp.int32),
            pltpu.VMEM((P * 8, ROW_F), jnp.float32),
            pltpu.VMEM((P * 8, ROW_F), jnp.float32),
            pltpu.VMEM((P, OUT_F), jnp.float32),
            pltpu.SemaphoreType.DMA,
            pltpu.SemaphoreType.DMA,
        ],
    )


def kernel(in_tensor, conditioning_code, tables):
    table_t = jnp.transpose(tables, (1, 2, 0, 3)).reshape(N_LEVELS * T, ROW_F)
    return _build_sc_kernel()(in_tensor, conditioning_code, table_t)
